# baseline (speedup 1.0000x reference)
import functools

import jax
import jax.numpy as jnp
from jax import lax
from jax.experimental import pallas as pl
from jax.experimental.pallas import tpu as pltpu
from jax.experimental.pallas import tpu_sc as plsc

_N = 10000
_E = 320000
_F = 128

_NC = 2
_NS = 16
_NW = _NC * _NS

_EPT = _E // _NW
_CHUNK = 40
_NP = 10240
_ROWS_PT = _NP // _NS
_ZR = _CHUNK
_ZSTEPS = _ROWS_PT // _ZR

_BLK = 2000
_NBLK = _EPT // _BLK
_CPB = _BLK // _CHUNK
_P = 6
_KST = _CPB // _P - 1


def _spmm3_body(zr, x0, x1, x2, e0, e1, e2, out, *scr):
    dblks = (scr[0], scr[2])
    sblks = (scr[1], scr[3])
    rows = scr[4:4 + _P]
    acc = scr[4 + _P]
    sems = scr[5 + _P:5 + 2 * _P]
    semid = scr[5 + 2 * _P]
    semis = scr[6 + 2 * _P]

    c = lax.axis_index("c")
    s = lax.axis_index("s")
    wid = c * _NS + s
    ebase = wid * _EPT
    rbase = s * _ROWS_PT

    pltpu.sync_copy(zr, acc.at[pl.ds(rbase, _ROWS_PT)])
    plsc.subcore_barrier()

    dirs = ((x0, e0), (x1, e1), (x2, e2))

    def _gather_start(xh, sblk, i, b):
        pltpu.async_copy(xh.at[sblk.at[pl.ds(i * _CHUNK, _CHUNK)]],
                         rows[b], sems[b])

    def _gather_wait(xh, sblk, i, b):
        pltpu.make_async_copy(xh.at[sblk.at[pl.ds(i * _CHUNK, _CHUNK)]],
                              rows[b], sems[b]).wait()

    def _scatter(dblk, i, b):
        pltpu.sync_copy(rows[b], acc.at[dblk.at[pl.ds(i * _CHUNK, _CHUNK)]],
                        add=True)

    def _head(d):
        xh, eh = dirs[d]
        pltpu.sync_copy(eh.at[pl.ds(ebase, _BLK)], dblks[0])
        pltpu.sync_copy(eh.at[pl.ds(_E + ebase, _BLK)], sblks[0])
        for q in range(_P - 1):
            _gather_start(xh, sblks[0], q, q)

    _head(0)

    for d in range(3):
        xh, eh = dirs[d]

        for b in range(_NBLK):
            dblk, sblk = dblks[b % 2], sblks[b % 2]
            if b + 1 < _NBLK:
                off = ebase + (b + 1) * _BLK
                pltpu.async_copy(eh.at[pl.ds(off, _BLK)], dblks[(b + 1) % 2],
                                 semid)
                pltpu.async_copy(eh.at[pl.ds(_E + off, _BLK)],
                                 sblks[(b + 1) % 2], semis)

            if b > 0:
                for q in range(_P - 1):
                    _gather_start(xh, sblk, q, q)

            def _ring(k, carry, dblk=dblk, sblk=sblk):
                i = k * _P
                _gather_start(xh, sblk, i + _P - 1, _P - 1)
                for j in range(_P):
                    _gather_wait(xh, sblk, i + j, j)
                    _scatter(dblk, i + j, j)
                    if j < _P - 1:
                        _gather_start(xh, sblk, i + j + _P, j)
                return carry

            lax.fori_loop(0, _KST, _ring, 0)

            e = _KST * _P
            _gather_start(xh, sblk, e + _P - 1, _P - 1)
            for i in range(e, _CPB):
                _gather_wait(xh, sblk, i, i % _P)
                _scatter(dblk, i, i % _P)
                if i + _P < _CPB:
                    _gather_start(xh, sblk, i + _P, i % _P)

            if b + 1 < _NBLK:
                off = ebase + (b + 1) * _BLK
                pltpu.make_async_copy(eh.at[pl.ds(off, _BLK)],
                                      dblks[(b + 1) % 2], semid).wait()
                pltpu.make_async_copy(eh.at[pl.ds(_E + off, _BLK)],
                                      sblks[(b + 1) % 2], semis).wait()

        plsc.subcore_barrier()

        if d < 2:
            _head(d + 1)
        pltpu.sync_copy(acc.at[pl.ds(rbase, _ROWS_PT)],
                        out.at[d, c, pl.ds(rbase, _ROWS_PT)])
        if d < 2:
            plsc.subcore_barrier()


_spmm3 = functools.partial(
    pl.kernel,
    out_type=jax.ShapeDtypeStruct((3, _NC, _NP, _F), jnp.float32),
    mesh=plsc.VectorSubcoreMesh(core_axis_name="c", subcore_axis_name="s"),
    scratch_types=(
        [pltpu.VMEM((_BLK,), jnp.int32)] * 4
        + [pltpu.VMEM((_CHUNK, _F), jnp.float32)] * _P
        + [pltpu.VMEM_SHARED((_NP, _F), jnp.float32)]
        + [pltpu.SemaphoreType.DMA] * (_P + 2)
    ),
)(_spmm3_body)


_BR = 5000


def _lin1_body(u_ref, i_ref, o_ref, w_ref, b_ref, xu_ref, xi_ref, xo_ref):
    w = w_ref[...]
    b = b_ref[...]
    xu_ref[...] = jnp.dot(u_ref[...], w, preferred_element_type=jnp.float32) + b
    xi_ref[...] = jnp.dot(i_ref[...], w, preferred_element_type=jnp.float32) + b
    xo_ref[...] = jnp.dot(o_ref[...], w, preferred_element_type=jnp.float32) + b


def _lin1(u, i, o, w, b):
    bs_x = pl.BlockSpec((_BR, _F), lambda g: (g, 0))
    bs_w = pl.BlockSpec((_F, _F), lambda g: (0, 0))
    bs_b = pl.BlockSpec((1, _F), lambda g: (0, 0))
    return pl.pallas_call(
        _lin1_body,
        grid=(_N // _BR,),
        in_specs=[bs_x, bs_x, bs_x, bs_w, bs_b],
        out_specs=[bs_x, bs_x, bs_x],
        out_shape=[jax.ShapeDtypeStruct((_N, _F), jnp.float32)] * 3,
    )(u, i, o, w, b.reshape(1, _F))


def _combine_body(p_ref, w_ref, b_ref, o_ref):
    acc = b_ref[...]
    prev = jnp.zeros_like(p_ref[0, 0])
    for d in range(3):
        cur = p_ref[d, 0] + p_ref[d, 1]
        xd = jnp.maximum(cur - prev, 0.0)
        prev = cur
        acc = acc + jnp.dot(xd, w_ref[d * _F:(d + 1) * _F, :],
                            preferred_element_type=jnp.float32)
    o_ref[...] = acc


def _combine(p, w, b):
    fout = w.shape[1]
    return pl.pallas_call(
        _combine_body,
        grid=(_N // _BR,),
        in_specs=[
            pl.BlockSpec((3, _NC, _BR, _F), lambda g: (0, 0, g, 0)),
            pl.BlockSpec((3 * _F, fout), lambda g: (0, 0)),
            pl.BlockSpec((1, fout), lambda g: (0, 0)),
        ],
        out_specs=pl.BlockSpec((_BR, fout), lambda g: (g, 0)),
        out_shape=jax.ShapeDtypeStruct((_N, fout), jnp.float32),
    )(p, w, b.reshape(1, fout))


def kernel(un_feature, in_feature, out_feature, un_edge_index, in_edge_index,
           out_edge_index, W1, b1, W2, b2, W3, b3):
    e_un = un_edge_index.reshape(-1)
    e_in = in_edge_index.reshape(-1)
    e_out = out_edge_index.reshape(-1)

    zr = jnp.zeros((_ROWS_PT, _F), jnp.float32)
    xu, xi, xo = _lin1(un_feature, in_feature, out_feature, W1, b1)
    p1 = _spmm3(zr, xu, xi, xo, e_un, e_in, e_out)
    x = _combine(p1, W2, b2)
    p2 = _spmm3(zr, x, x, x, e_un, e_in, e_out)
    return _combine(p2, W3, b3)


# speedup vs baseline: 1.0693x; 1.0693x over previous
import functools

import jax
import jax.numpy as jnp
from jax import lax
from jax.experimental import pallas as pl
from jax.experimental.pallas import tpu as pltpu
from jax.experimental.pallas import tpu_sc as plsc

_N = 10000
_E = 320000
_F = 128

_NC = 2
_NS = 16
_NW = _NC * _NS

_EPT = _E // _NW
_CHUNK = 40
_NP = 10240
_ROWS_PT = _NP // _NS
_ZR = _CHUNK
_ZSTEPS = _ROWS_PT // _ZR

_BLK = 2000
_NBLK = _EPT // _BLK
_CPB = _BLK // _CHUNK
_P = 5
_KST = (_CPB - _P) // _P


def _spmm3_body(zr, x0, x1, x2, e0, e1, e2, out, *scr):
    dblks = (scr[0], scr[2])
    sblks = (scr[1], scr[3])
    rows = scr[4:4 + _P]
    acc = scr[4 + _P]
    sems = scr[5 + _P:5 + 2 * _P]
    semid = scr[5 + 2 * _P]
    semis = scr[6 + 2 * _P]

    c = lax.axis_index("c")
    s = lax.axis_index("s")
    wid = c * _NS + s
    ebase = wid * _EPT
    rbase = s * _ROWS_PT
    dirs = ((x0, e0), (x1, e1), (x2, e2))
    nglob = 3 * _NBLK

    def _gather_start(xh, sblk, i, b):
        pltpu.async_copy(xh.at[sblk.at[pl.ds(i * _CHUNK, _CHUNK)]],
                         rows[b], sems[b])

    def _gather_wait(xh, sblk, i, b):
        pltpu.make_async_copy(xh.at[sblk.at[pl.ds(i * _CHUNK, _CHUNK)]],
                              rows[b], sems[b]).wait()

    def _scatter(dblk, i, b):
        pltpu.sync_copy(rows[b], acc.at[dblk.at[pl.ds(i * _CHUNK, _CHUNK)]],
                        add=True)

    pltpu.sync_copy(zr, acc.at[pl.ds(rbase, _ROWS_PT)])
    plsc.subcore_barrier()

    pltpu.sync_copy(e0.at[pl.ds(ebase, _BLK)], dblks[0])
    pltpu.sync_copy(e0.at[pl.ds(_E + ebase, _BLK)], sblks[0])
    for q in range(_P):
        _gather_start(x0, sblks[0], q, q)

    for g in range(nglob):
        d, b = divmod(g, _NBLK)
        xh, eh = dirs[d]
        par = g % 2
        dblk, sblk = dblks[par], sblks[par]

        if g + 1 < nglob:
            dn, bn = divmod(g + 1, _NBLK)
            ehn = dirs[dn][1]
            offn = ebase + bn * _BLK
            pltpu.async_copy(ehn.at[pl.ds(offn, _BLK)], dblks[par ^ 1], semid)
            pltpu.async_copy(ehn.at[pl.ds(_E + offn, _BLK)], sblks[par ^ 1],
                             semis)

        def _ring(k, carry, dblk=dblk, sblk=sblk, xh=xh):
            i = k * _P
            for j in range(_P):
                q = i + j
                _gather_wait(xh, sblk, q, j)
                _scatter(dblk, q, j)
                _gather_start(xh, sblk, q + _P, j)
            return carry

        lax.fori_loop(0, _KST, _ring, 0)

        if g + 1 < nglob:
            dn, bn = divmod(g + 1, _NBLK)
            ehn = dirs[dn][1]
            offn = ebase + bn * _BLK
            pltpu.make_async_copy(ehn.at[pl.ds(offn, _BLK)], dblks[par ^ 1],
                                  semid).wait()
            pltpu.make_async_copy(ehn.at[pl.ds(_E + offn, _BLK)],
                                  sblks[par ^ 1], semis).wait()

        for q in range(_CPB - _P, _CPB):
            _gather_wait(xh, sblk, q, q % _P)
            _scatter(dblk, q, q % _P)
            if g + 1 < nglob:
                xhn = dirs[divmod(g + 1, _NBLK)[0]][0]
                _gather_start(xhn, sblks[par ^ 1], q + _P - _CPB, q % _P)

        if b == _NBLK - 1:
            plsc.subcore_barrier()
            pltpu.sync_copy(acc.at[pl.ds(rbase, _ROWS_PT)],
                            out.at[d, c, pl.ds(rbase, _ROWS_PT)])
            if d < 2:
                plsc.subcore_barrier()


_spmm3 = functools.partial(
    pl.kernel,
    out_type=jax.ShapeDtypeStruct((3, _NC, _NP, _F), jnp.float32),
    mesh=plsc.VectorSubcoreMesh(core_axis_name="c", subcore_axis_name="s"),
    scratch_types=(
        [pltpu.VMEM((_BLK,), jnp.int32)] * 4
        + [pltpu.VMEM((_CHUNK, _F), jnp.float32)] * _P
        + [pltpu.VMEM_SHARED((_NP, _F), jnp.float32)]
        + [pltpu.SemaphoreType.DMA] * (_P + 2)
    ),
)(_spmm3_body)


_BR = 2000


def _lin1_body(u_ref, i_ref, o_ref, w_ref, b_ref, xu_ref, xi_ref, xo_ref):
    w = w_ref[...]
    b = b_ref[...]
    xu_ref[...] = jnp.dot(u_ref[...], w, preferred_element_type=jnp.float32) + b
    xi_ref[...] = jnp.dot(i_ref[...], w, preferred_element_type=jnp.float32) + b
    xo_ref[...] = jnp.dot(o_ref[...], w, preferred_element_type=jnp.float32) + b


def _lin1(u, i, o, w, b):
    bs_x = pl.BlockSpec((_BR, _F), lambda g: (g, 0))
    bs_w = pl.BlockSpec((_F, _F), lambda g: (0, 0))
    bs_b = pl.BlockSpec((1, _F), lambda g: (0, 0))
    return pl.pallas_call(
        _lin1_body,
        grid=(_N // _BR,),
        in_specs=[bs_x, bs_x, bs_x, bs_w, bs_b],
        out_specs=[bs_x, bs_x, bs_x],
        out_shape=[jax.ShapeDtypeStruct((_N, _F), jnp.float32)] * 3,
    )(u, i, o, w, b.reshape(1, _F))


def _combine_body(p_ref, w_ref, b_ref, o_ref):
    acc = b_ref[...]
    prev = jnp.zeros_like(p_ref[0, 0])
    for d in range(3):
        cur = p_ref[d, 0] + p_ref[d, 1]
        xd = jnp.maximum(cur - prev, 0.0)
        prev = cur
        acc = acc + jnp.dot(xd, w_ref[d * _F:(d + 1) * _F, :],
                            preferred_element_type=jnp.float32)
    o_ref[...] = acc


def _combine(p, w, b):
    fout = w.shape[1]
    return pl.pallas_call(
        _combine_body,
        grid=(_N // _BR,),
        in_specs=[
            pl.BlockSpec((3, _NC, _BR, _F), lambda g: (0, 0, g, 0)),
            pl.BlockSpec((3 * _F, fout), lambda g: (0, 0)),
            pl.BlockSpec((1, fout), lambda g: (0, 0)),
        ],
        out_specs=pl.BlockSpec((_BR, fout), lambda g: (g, 0)),
        out_shape=jax.ShapeDtypeStruct((_N, fout), jnp.float32),
    )(p, w, b.reshape(1, fout))


def kernel(un_feature, in_feature, out_feature, un_edge_index, in_edge_index,
           out_edge_index, W1, b1, W2, b2, W3, b3):
    e_un = un_edge_index.reshape(-1)
    e_in = in_edge_index.reshape(-1)
    e_out = out_edge_index.reshape(-1)

    zr = jnp.zeros((_ROWS_PT, _F), jnp.float32)
    xu, xi, xo = _lin1(un_feature, in_feature, out_feature, W1, b1)
    p1 = _spmm3(zr, xu, xi, xo, e_un, e_in, e_out)
    x = _combine(p1, W2, b2)
    p2 = _spmm3(zr, x, x, x, e_un, e_in, e_out)
    return _combine(p2, W3, b3)
